# baseline (speedup 1.0000x reference)
import functools

import jax
import jax.numpy as jnp
from jax import lax
from jax.experimental import pallas as pl
from jax.experimental.pallas import tpu as pltpu
from jax.experimental.pallas import tpu_sc as plsc

N_NODES = 10000
N_EDGES = 320000
D = 128

NC = 2
NS = 16
L = 16
NW = NC * NS
EPW = N_EDGES // NW
BLK = 80
NBLK = EPW // BLK
RCH = N_NODES // BLK

_mesh = plsc.VectorSubcoreMesh(core_axis_name="c", subcore_axis_name="s")

_sc_params = pltpu.CompilerParams()
if "needs_layout_passes" in pltpu.CompilerParams.__dataclass_fields__:
    import dataclasses as _dc
    _sc_params = _dc.replace(_sc_params, needs_layout_passes=False)


def _zero_buf(buf):
    @pl.loop(0, BLK)
    def _(e):
        for cc in range(buf.shape[1] // L):
            buf[e, pl.ds(cc * L, L)] = jnp.zeros((L,), jnp.float32)


def _make_sc_agg(with_wsum):
    def body(x_hbm, src_hbm, dst_hbm, w_hbm, *rest):
        if with_wsum:
            out_hbm, ws_hbm = rest[0], rest[1]
            scr = rest[2:]
        else:
            out_hbm = rest[0]
            scr = rest[1:]
        (buf0, buf1, sb0, db0, sb1, db1, sb2, db2, sb3, db3,
         wv0, wv1, wv2, wv3, gs0, gs1, ss0, ss1, ps0, ps1, ps2, ps3) = \
            scr[:22]
        accw = scr[22] if with_wsum else None
        acc = scr[-1]

        c = lax.axis_index("c")
        s = lax.axis_index("s")
        wid = s * NC + c

        bufs = (buf0, buf1)
        gsems = (gs0, gs1)
        ssems = (ss0, ss1)
        sbs = (sb0, sb1, sb2, sb3)
        dbs = (db0, db1, db2, db3)
        wvs = (wv0, wv1, wv2, wv3)
        psems = (ps0, ps1, ps2, ps3)

        def prep(b, i):
            pltpu.async_copy(src_hbm.at[wid, b], sbs[i], psems[i])
            pltpu.async_copy(dst_hbm.at[wid, b], dbs[i], psems[i])
            pltpu.async_copy(w_hbm.at[wid, b], wvs[i], psems[i])

        def issue_gather(b_j, i):
            pltpu.make_async_copy(src_hbm.at[0, 0], sbs[i], psems[i]).wait()
            pltpu.make_async_copy(dst_hbm.at[0, 0], dbs[i], psems[i]).wait()
            pltpu.make_async_copy(w_hbm.at[0, 0], wvs[i], psems[i]).wait()
            pltpu.async_copy(x_hbm.at[dbs[i]], bufs[b_j], gsems[b_j])

        def scale(buf, wv):
            @plsc.parallel_loop(0, BLK, unroll=4)
            def _(e):
                we = plsc.load_gather(wv, [jnp.full((L,), e, jnp.int32)])
                for cc in range(D // L):
                    sl = (e, pl.ds(cc * L, L))
                    buf[sl] = buf[sl] * we

        def compute(b_j, i):
            pltpu.make_async_copy(x_hbm.at[pl.ds(0, BLK)], bufs[b_j],
                                  gsems[b_j]).wait()
            scale(bufs[b_j], wvs[i])
            pltpu.async_copy(bufs[b_j], acc.at[sbs[i]], ssems[b_j], add=True)
            if with_wsum:
                for j in range(BLK // L):
                    sl = pl.ds(j * L, L)
                    plsc.addupdate_scatter(accw, [sbs[i][sl]], wvs[i][sl])

        def drain_scatter(b_j, i):
            pltpu.make_async_copy(bufs[b_j], acc.at[sbs[i]], ssems[b_j]).wait()

        if with_wsum:
            @pl.loop(0, N_NODES // L)
            def _(i):
                accw[pl.ds(i * L, L)] = jnp.zeros((L,), jnp.float32)

        _zero_buf(buf0)
        for j in range((RCH + NS - 1) // NS):
            ch = s + NS * j

            @pl.when(ch < RCH)
            def _():
                pltpu.sync_copy(buf0, acc.at[pl.ds(ch * BLK, BLK)])

        plsc.subcore_barrier()

        prep(0, 0)
        prep(1, 1)
        prep(2, 2)
        issue_gather(0, 0)

        @pl.loop(0, NBLK - 1, step=4)
        def _(k):
            for m in range(0, 4, 2):
                b = k + m
                j0, j1 = m % 2, (m + 1) % 2
                i0, i1, i2, i3 = m, (m + 1) % 4, (m + 2) % 4, (m + 3) % 4
                if m == 0:
                    @pl.when(k > 0)
                    def _():
                        drain_scatter(1, 3)
                else:
                    drain_scatter(j1, i3)
                issue_gather(j1, i1)

                compute(j0, i0)
                @pl.when(b + 3 < NBLK)
                def _():
                    prep(b + 3, i3)

                compute(j1, i1)
                drain_scatter(j0, i0)
                issue_gather(j0, i2)
                @pl.when(b + 4 < NBLK)
                def _():
                    prep(b + 4, i0)

        compute(0, 0)
        drain_scatter(1, 3)
        drain_scatter(0, 0)

        if with_wsum:
            pltpu.sync_copy(accw, ws_hbm.at[wid])

        plsc.subcore_barrier()
        for j in range((RCH + NS - 1) // NS):
            ch = s + NS * j

            @pl.when(ch < RCH)
            def _():
                pltpu.sync_copy(acc.at[pl.ds(ch * BLK, BLK)],
                                out_hbm.at[c].at[pl.ds(ch * BLK, BLK)])

    agg_out = jax.ShapeDtypeStruct((NC, N_NODES, D), jnp.float32)
    ws_out = jax.ShapeDtypeStruct((NW, N_NODES), jnp.float32)
    return pl.kernel(
        body,
        out_type=[agg_out, ws_out] if with_wsum else agg_out,
        mesh=_mesh,
        scratch_types=(
            [pltpu.VMEM((BLK, D), jnp.float32)] * 2
            + [pltpu.VMEM((BLK,), jnp.int32)] * 8
            + [pltpu.VMEM((BLK,), jnp.float32)] * 4
            + [pltpu.SemaphoreType.DMA] * 8
            + ([pltpu.VMEM((N_NODES,), jnp.float32)] if with_wsum else [])
            + [pltpu.VMEM_SHARED((N_NODES, D), jnp.float32)]
        ),
        compiler_params=_sc_params,
    )


_sc_agg_ws = _make_sc_agg(True)
_sc_agg = _make_sc_agg(False)


BR = 10000


def _mm_t(a, w):
    return jax.lax.dot_general(a, w, (((1,), (1,)), ((), ())),
                               preferred_element_type=jnp.float32)


def _tc_layer1_body(x_ref, p0_ref, p1_ref, wp_ref, w_ref, b_ref,
                    h_ref, ws_ref):
    ws = jnp.clip(jnp.sum(wp_ref[0], axis=0), 1e-12, None)
    neigh = (p0_ref[...] + p1_ref[...]) / ws[:, None]
    h = _mm_t(x_ref[...], w_ref[:, :D]) + _mm_t(neigh, w_ref[:, D:])
    h = h + b_ref[...]
    h_ref[...] = jnp.maximum(h, 0.0)
    ws_ref[...] = ws[None, None, :]


def _tc_layer2_body(x_ref, p0_ref, p1_ref, ws_ref, w_ref, b_ref,
                    o_ref):
    ws = ws_ref[0, 0]
    neigh = (p0_ref[...] + p1_ref[...]) / ws[:, None]
    h = _mm_t(x_ref[...], w_ref[:, :D]) + _mm_t(neigh, w_ref[:, D:])
    h = h + b_ref[...]
    h = jnp.maximum(h, 0.0)
    nrm = jnp.sqrt(jnp.sum(h * h, axis=1, keepdims=True))
    o_ref[...] = h / jnp.clip(nrm, 1e-12, None)


NBR = N_NODES // BR

_row_spec = pl.BlockSpec((BR, D), lambda i: (i, 0))
_full_w = pl.BlockSpec((D, 2 * D), lambda i: (0, 0))
_bias_spec = pl.BlockSpec((1, D), lambda i: (0, 0))
_ws_spec = pl.BlockSpec((1, 1, BR), lambda i: (i, 0, 0))

_tc_layer1 = pl.pallas_call(
    _tc_layer1_body,
    grid=(NBR,),
    in_specs=[_row_spec, _row_spec, _row_spec,
              pl.BlockSpec((1, NW, BR), lambda i: (i, 0, 0)),
              _full_w, _bias_spec],
    out_specs=[_row_spec, _ws_spec],
    out_shape=[jax.ShapeDtypeStruct((N_NODES, D), jnp.float32),
               jax.ShapeDtypeStruct((NBR, 1, N_NODES // NBR), jnp.float32)],
)

_tc_layer2 = pl.pallas_call(
    _tc_layer2_body,
    grid=(N_NODES // BR,),
    in_specs=[_row_spec, _row_spec, _row_spec, _ws_spec,
              _full_w, _bias_spec],
    out_specs=_row_spec,
    out_shape=jax.ShapeDtypeStruct((N_NODES, D), jnp.float32),
)


def kernel(x, edge_index, edge_weight, W1, b1, W2, b2):
    src_b = edge_index[0].astype(jnp.int32).reshape(NW, NBLK, BLK)
    dst_b = edge_index[1].astype(jnp.int32).reshape(NW, NBLK, BLK)
    w_b = edge_weight.astype(jnp.float32).reshape(NW, NBLK, BLK)

    b1r = b1.reshape(1, D)
    b2r = b2.reshape(1, D)

    p, wpart = _sc_agg_ws(x, src_b, dst_b, w_b)
    wpart = wpart.reshape(NW, NBR, BR).transpose(1, 0, 2)
    h1, ws = _tc_layer1(x, p[0], p[1], wpart, W1, b1r)
    q = _sc_agg(h1, src_b, dst_b, w_b)
    out = _tc_layer2(h1, q[0], q[1], ws, W2, b2r)
    return out


# speedup vs baseline: 1.0037x; 1.0037x over previous
import functools

import jax
import jax.numpy as jnp
from jax import lax
from jax.experimental import pallas as pl
from jax.experimental.pallas import tpu as pltpu
from jax.experimental.pallas import tpu_sc as plsc

N_NODES = 10000
N_EDGES = 320000
D = 128

NC = 2
NS = 16
L = 16
NW = NC * NS
EPW = N_EDGES // NW
BLK = 80
NBLK = EPW // BLK
RCH = N_NODES // BLK

_mesh = plsc.VectorSubcoreMesh(core_axis_name="c", subcore_axis_name="s")

_sc_params = pltpu.CompilerParams()
if "needs_layout_passes" in pltpu.CompilerParams.__dataclass_fields__:
    import dataclasses as _dc
    _sc_params = _dc.replace(_sc_params, needs_layout_passes=False)


def _zero_buf(buf):
    @pl.loop(0, BLK)
    def _(e):
        for cc in range(buf.shape[1] // L):
            buf[e, pl.ds(cc * L, L)] = jnp.zeros((L,), jnp.float32)


def _make_sc_agg(with_wsum):
    def body(x_hbm, src_hbm, dst_hbm, w_hbm, *rest):
        if with_wsum:
            out_hbm, ws_hbm = rest[0], rest[1]
            scr = rest[2:]
        else:
            out_hbm = rest[0]
            scr = rest[1:]
        (buf0, buf1, sb0, db0, sb1, db1, sb2, db2, sb3, db3,
         wv0, wv1, wv2, wv3, gs0, gs1, ss0, ss1, ps0, ps1, ps2, ps3) = \
            scr[:22]
        accw = scr[22] if with_wsum else None
        acc = scr[-1]

        c = lax.axis_index("c")
        s = lax.axis_index("s")
        wid = s * NC + c

        bufs = (buf0, buf1)
        gsems = (gs0, gs1)
        ssems = (ss0, ss1)
        sbs = (sb0, sb1, sb2, sb3)
        dbs = (db0, db1, db2, db3)
        wvs = (wv0, wv1, wv2, wv3)
        psems = (ps0, ps1, ps2, ps3)

        def prep(b, i):
            pltpu.async_copy(src_hbm.at[wid, b], sbs[i], psems[i])
            pltpu.async_copy(dst_hbm.at[wid, b], dbs[i], psems[i])
            pltpu.async_copy(w_hbm.at[wid, b], wvs[i], psems[i])

        def issue_gather(b_j, i):
            pltpu.make_async_copy(src_hbm.at[0, 0], sbs[i], psems[i]).wait()
            pltpu.make_async_copy(dst_hbm.at[0, 0], dbs[i], psems[i]).wait()
            pltpu.make_async_copy(w_hbm.at[0, 0], wvs[i], psems[i]).wait()
            pltpu.async_copy(x_hbm.at[dbs[i]], bufs[b_j], gsems[b_j])

        def scale(buf, wv):
            @plsc.parallel_loop(0, BLK, unroll=4)
            def _(e):
                we = plsc.load_gather(wv, [jnp.full((L,), e, jnp.int32)])
                for cc in range(D // L):
                    sl = (e, pl.ds(cc * L, L))
                    buf[sl] = buf[sl] * we

        def compute(b_j, i):
            pltpu.make_async_copy(x_hbm.at[pl.ds(0, BLK)], bufs[b_j],
                                  gsems[b_j]).wait()
            scale(bufs[b_j], wvs[i])
            pltpu.async_copy(bufs[b_j], acc.at[sbs[i]], ssems[b_j], add=True)
            if with_wsum:
                for j in range(BLK // L):
                    sl = pl.ds(j * L, L)
                    plsc.addupdate_scatter(accw, [sbs[i][sl]], wvs[i][sl])

        def drain_scatter(b_j, i):
            pltpu.make_async_copy(bufs[b_j], acc.at[sbs[i]], ssems[b_j]).wait()

        prep(0, 0)
        prep(1, 1)
        prep(2, 2)
        issue_gather(0, 0)

        if with_wsum:
            @pl.loop(0, N_NODES // L)
            def _(i):
                accw[pl.ds(i * L, L)] = jnp.zeros((L,), jnp.float32)

        _zero_buf(buf1)
        for j in range((RCH + NS - 1) // NS):
            ch = s + NS * j

            @pl.when(ch < RCH)
            def _():
                pltpu.sync_copy(buf1, acc.at[pl.ds(ch * BLK, BLK)])

        plsc.subcore_barrier()


        @pl.loop(0, NBLK - 1, step=4)
        def _(k):
            for m in range(0, 4, 2):
                b = k + m
                j0, j1 = m % 2, (m + 1) % 2
                i0, i1, i2, i3 = m, (m + 1) % 4, (m + 2) % 4, (m + 3) % 4
                if m == 0:
                    @pl.when(k > 0)
                    def _():
                        drain_scatter(1, 3)
                else:
                    drain_scatter(j1, i3)
                issue_gather(j1, i1)

                compute(j0, i0)
                @pl.when(b + 3 < NBLK)
                def _():
                    prep(b + 3, i3)

                compute(j1, i1)
                drain_scatter(j0, i0)
                issue_gather(j0, i2)
                @pl.when(b + 4 < NBLK)
                def _():
                    prep(b + 4, i0)

        compute(0, 0)
        drain_scatter(1, 3)
        drain_scatter(0, 0)

        if with_wsum:
            pltpu.sync_copy(accw, ws_hbm.at[wid])

        plsc.subcore_barrier()
        for j in range((RCH + NS - 1) // NS):
            ch = s + NS * j

            @pl.when(ch < RCH)
            def _():
                pltpu.sync_copy(acc.at[pl.ds(ch * BLK, BLK)],
                                out_hbm.at[c].at[pl.ds(ch * BLK, BLK)])

    agg_out = jax.ShapeDtypeStruct((NC, N_NODES, D), jnp.float32)
    ws_out = jax.ShapeDtypeStruct((NW, N_NODES), jnp.float32)
    return pl.kernel(
        body,
        out_type=[agg_out, ws_out] if with_wsum else agg_out,
        mesh=_mesh,
        scratch_types=(
            [pltpu.VMEM((BLK, D), jnp.float32)] * 2
            + [pltpu.VMEM((BLK,), jnp.int32)] * 8
            + [pltpu.VMEM((BLK,), jnp.float32)] * 4
            + [pltpu.SemaphoreType.DMA] * 8
            + ([pltpu.VMEM((N_NODES,), jnp.float32)] if with_wsum else [])
            + [pltpu.VMEM_SHARED((N_NODES, D), jnp.float32)]
        ),
        compiler_params=_sc_params,
    )


_sc_agg_ws = _make_sc_agg(True)
_sc_agg = _make_sc_agg(False)


BR = 5000


def _mm_t(a, w):
    return jax.lax.dot_general(a, w, (((1,), (1,)), ((), ())),
                               preferred_element_type=jnp.float32)


def _tc_layer1_body(x_ref, p0_ref, p1_ref, wp_ref, w_ref, b_ref,
                    h_ref, ws_ref):
    ws = jnp.clip(jnp.sum(wp_ref[0], axis=0), 1e-12, None)
    neigh = (p0_ref[...] + p1_ref[...]) / ws[:, None]
    h = _mm_t(x_ref[...], w_ref[:, :D]) + _mm_t(neigh, w_ref[:, D:])
    h = h + b_ref[...]
    h_ref[...] = jnp.maximum(h, 0.0)
    ws_ref[...] = ws[None, None, :]


def _tc_layer2_body(x_ref, p0_ref, p1_ref, ws_ref, w_ref, b_ref,
                    o_ref):
    ws = ws_ref[0, 0]
    neigh = (p0_ref[...] + p1_ref[...]) / ws[:, None]
    h = _mm_t(x_ref[...], w_ref[:, :D]) + _mm_t(neigh, w_ref[:, D:])
    h = h + b_ref[...]
    h = jnp.maximum(h, 0.0)
    nrm = jnp.sqrt(jnp.sum(h * h, axis=1, keepdims=True))
    o_ref[...] = h / jnp.clip(nrm, 1e-12, None)


NBR = N_NODES // BR

_row_spec = pl.BlockSpec((BR, D), lambda i: (i, 0))
_full_w = pl.BlockSpec((D, 2 * D), lambda i: (0, 0))
_bias_spec = pl.BlockSpec((1, D), lambda i: (0, 0))
_ws_spec = pl.BlockSpec((1, 1, BR), lambda i: (i, 0, 0))

_tc_layer1 = pl.pallas_call(
    _tc_layer1_body,
    grid=(NBR,),
    in_specs=[_row_spec, _row_spec, _row_spec,
              pl.BlockSpec((1, NW, BR), lambda i: (i, 0, 0)),
              _full_w, _bias_spec],
    out_specs=[_row_spec, _ws_spec],
    out_shape=[jax.ShapeDtypeStruct((N_NODES, D), jnp.float32),
               jax.ShapeDtypeStruct((NBR, 1, N_NODES // NBR), jnp.float32)],
)

_tc_layer2 = pl.pallas_call(
    _tc_layer2_body,
    grid=(N_NODES // BR,),
    in_specs=[_row_spec, _row_spec, _row_spec, _ws_spec,
              _full_w, _bias_spec],
    out_specs=_row_spec,
    out_shape=jax.ShapeDtypeStruct((N_NODES, D), jnp.float32),
)


def kernel(x, edge_index, edge_weight, W1, b1, W2, b2):
    src_b = edge_index[0].astype(jnp.int32).reshape(NW, NBLK, BLK)
    dst_b = edge_index[1].astype(jnp.int32).reshape(NW, NBLK, BLK)
    w_b = edge_weight.astype(jnp.float32).reshape(NW, NBLK, BLK)

    b1r = b1.reshape(1, D)
    b2r = b2.reshape(1, D)

    p, wpart = _sc_agg_ws(x, src_b, dst_b, w_b)
    wpart = wpart.reshape(NW, NBR, BR).transpose(1, 0, 2)
    h1, ws = _tc_layer1(x, p[0], p[1], wpart, W1, b1r)
    q = _sc_agg(h1, src_b, dst_b, w_b)
    out = _tc_layer2(h1, q[0], q[1], ws, W2, b2r)
    return out
